# Initial kernel scaffold; baseline (speedup 1.0000x reference)
import jax
import jax.numpy as jnp
from jax import lax
from jax.experimental import pallas as pl
from jax.experimental.pallas import tpu as pltpu
from jax.experimental.pallas import tpu_sc as plsc

NUM_MODELS = 8
NUM_BOXES = 100000
DIM = 64
BATCH = 4096

_INFO = plsc.get_sparse_core_info()
NC = _INFO.num_cores
NS = _INFO.num_subcores
LANES = _INFO.num_lanes
NW = NC * NS
BPW = BATCH // NW


def _sc_body(z_hbm, ld_hbm, idx_hbm, out_hbm,
             idx_v, idxo_v, z_v, ld_v, sem_z, sem_l):
    cid = lax.axis_index("c")
    sid = lax.axis_index("s")
    wid = sid * NC + cid
    base = wid * BPW

    pltpu.sync_copy(idx_hbm.at[pl.ds(base, BPW)], idx_v)

    def model_body(m, carry):
        off = m * NUM_BOXES
        for i in range(BPW // LANES):
            sl = pl.ds(i * LANES, LANES)
            idxo_v[sl] = idx_v[sl] + off

        cp_z = pltpu.make_async_copy(z_hbm.at[idxo_v], z_v, sem_z)
        cp_l = pltpu.make_async_copy(ld_hbm.at[idxo_v], ld_v, sem_l)
        cp_z.start()
        cp_l.start()
        cp_z.wait()
        cp_l.wait()

        def row_body(r, c2):
            for cck in range(DIM // LANES):
                sl = pl.ds(cck * LANES, LANES)
                ld_v[r, sl] = z_v[r, sl] + jnp.exp(ld_v[r, sl])
            return c2

        lax.fori_loop(0, BPW, row_body, 0, unroll=2)

        row0 = m * BATCH + base
        pltpu.sync_copy(z_v, out_hbm.at[pl.ds(row0, BPW), pl.ds(0, DIM)])
        pltpu.sync_copy(ld_v, out_hbm.at[pl.ds(row0, BPW), pl.ds(DIM, DIM)])
        return carry

    lax.fori_loop(0, NUM_MODELS, model_body, 0)


@jax.jit
def kernel(box_indices, z, logdelta):
    z2 = z.reshape(NUM_MODELS * NUM_BOXES, DIM)
    ld2 = logdelta.reshape(NUM_MODELS * NUM_BOXES, DIM)
    idx = box_indices.astype(jnp.int32)

    mesh = plsc.VectorSubcoreMesh(core_axis_name="c", subcore_axis_name="s")
    out = pl.kernel(
        _sc_body,
        out_type=jax.ShapeDtypeStruct((NUM_MODELS * BATCH, 2 * DIM),
                                      jnp.float32),
        mesh=mesh,
        scratch_types=[
            pltpu.VMEM((BPW,), jnp.int32),
            pltpu.VMEM((BPW,), jnp.int32),
            pltpu.VMEM((BPW, DIM), jnp.float32),
            pltpu.VMEM((BPW, DIM), jnp.float32),
            pltpu.SemaphoreType.DMA,
            pltpu.SemaphoreType.DMA,
        ],
    )(z2, ld2, idx)

    return out.reshape(NUM_MODELS, BATCH, 2, DIM)


# speedup vs baseline: 1.4268x; 1.4268x over previous
import jax
import jax.numpy as jnp
from jax import lax
from jax.experimental import pallas as pl
from jax.experimental.pallas import tpu as pltpu
from jax.experimental.pallas import tpu_sc as plsc

NUM_MODELS = 8
NUM_BOXES = 100000
DIM = 64
BATCH = 4096

_INFO = plsc.get_sparse_core_info()
NC = _INFO.num_cores
NS = _INFO.num_subcores
LANES = _INFO.num_lanes
NW = NC * NS
NROWS = NUM_MODELS * DIM
RPW = NROWS // NW


def _sc_body(z_hbm, ld_hbm, idx_hbm, out_hbm, idx_v, row_v, zg_v, cap_v, sem):
    cid = lax.axis_index("c")
    sid = lax.axis_index("s")
    wid = sid * NC + cid
    r0 = wid * RPW

    pltpu.sync_copy(idx_hbm, idx_v)

    def do_row(i, carry):
        r = r0 + i
        m = r // DIM
        d = r % DIM

        pltpu.sync_copy(z_hbm.at[r], row_v)

        def zgather(j, c):
            sl = pl.ds(j * LANES, LANES)
            zg_v[sl] = plsc.load_gather(row_v, [idx_v[sl]])
            return c

        lax.fori_loop(0, BATCH // LANES, zgather, 0, unroll=4)
        pltpu.sync_copy(zg_v, out_hbm.at[m, 0, d])

        pltpu.sync_copy(ld_hbm.at[r], row_v)

        def lgather(j, c):
            sl = pl.ds(j * LANES, LANES)
            cap_v[sl] = zg_v[sl] + jnp.exp(
                plsc.load_gather(row_v, [idx_v[sl]]))
            return c

        lax.fori_loop(0, BATCH // LANES, lgather, 0, unroll=4)
        pltpu.sync_copy(cap_v, out_hbm.at[m, 1, d])
        return carry

    lax.fori_loop(0, RPW, do_row, 0)


@jax.jit
def kernel(box_indices, z, logdelta):
    zT = z.transpose(0, 2, 1).reshape(NROWS, NUM_BOXES)
    ldT = logdelta.transpose(0, 2, 1).reshape(NROWS, NUM_BOXES)
    idx = box_indices.astype(jnp.int32)

    mesh = plsc.VectorSubcoreMesh(core_axis_name="c", subcore_axis_name="s")
    out = pl.kernel(
        _sc_body,
        out_type=jax.ShapeDtypeStruct((NUM_MODELS, 2, DIM, BATCH),
                                      jnp.float32),
        mesh=mesh,
        compiler_params=pltpu.CompilerParams(needs_layout_passes=False),
        scratch_types=[
            pltpu.VMEM((BATCH,), jnp.int32),
            pltpu.VMEM((NUM_BOXES,), jnp.float32),
            pltpu.VMEM((BATCH,), jnp.float32),
            pltpu.VMEM((BATCH,), jnp.float32),
            pltpu.SemaphoreType.DMA,
        ],
    )(zT, ldT, idx)

    return out.transpose(0, 3, 1, 2)
